# Initial kernel scaffold; baseline (speedup 1.0000x reference)
import jax
import jax.numpy as jnp
from jax.experimental import pallas as pl


def kernel(feature, condition, edge_index, W_e1, b_e1, W_e2, b_e2, W_e3, b_e3, W_d1, b_d1, W_d2, b_d2, W_d3, b_d3):
    raise NotImplementedError("write your pallas kernel here")


# speedup vs baseline: 8.5906x; 8.5906x over previous
import functools

import jax
import jax.numpy as jnp
from jax import lax
from jax.experimental import pallas as pl
from jax.experimental.pallas import tpu as pltpu
from jax.experimental.pallas import tpu_sc as plsc

_N = 10000
_NP = 10240
_E = 320000
_NC = 2
_NS = 16
_NW = _NC * _NS
_EPW = _E // _NW
_K = 80
_NCHUNK = _EPW // _K
_RPW = _NP // _NS
_ZR = 128

_MESH = plsc.VectorSubcoreMesh(core_axis_name="c", subcore_axis_name="s")


def _fill(ref, rows, d, value):
    vec = jnp.full((16,), value, jnp.float32)
    cols = d // 16

    def body(i, carry):
        ref[i // cols, pl.ds((i % cols) * 16, 16)] = vec
        return carry

    lax.fori_loop(0, rows * cols, body, 0)


def _make_prop(d):
    @functools.partial(
        pl.kernel,
        out_type=jax.ShapeDtypeStruct((_NC * _NP, d), jnp.float32),
        mesh=_MESH,
        scratch_types=[
            pltpu.VMEM((_K,), jnp.int32),
            pltpu.VMEM((_K,), jnp.int32),
            pltpu.VMEM((_K, d), jnp.float32),
            pltpu.VMEM((_ZR, d), jnp.float32),
            pltpu.VMEM_SHARED((_NP, d), jnp.float32),
            pltpu.SemaphoreType.DMA,
        ],
    )
    def prop(y_hbm, src_hbm, dst_hbm, out_hbm, idx_s, idx_d, rows, zblk, acc, sem):
        c = lax.axis_index("c")
        s = lax.axis_index("s")
        wid = c * _NS + s

        _fill(zblk, _ZR, d, 0.0)
        rbase = s * _RPW
        for t in range(_RPW // _ZR):
            pltpu.sync_copy(zblk, acc.at[pl.ds(rbase + t * _ZR, _ZR)])
        plsc.subcore_barrier()

        ebase = wid * _EPW

        def body(j, carry):
            off = ebase + j * _K
            pltpu.sync_copy(src_hbm.at[pl.ds(off, _K)], idx_s)
            pltpu.sync_copy(dst_hbm.at[pl.ds(off, _K)], idx_d)
            pltpu.async_copy(y_hbm.at[idx_s], rows, sem).wait()
            pltpu.sync_copy(rows, acc.at[idx_d], add=True)
            return carry

        lax.fori_loop(0, _NCHUNK, body, 0)
        plsc.subcore_barrier()

        pltpu.sync_copy(acc.at[pl.ds(rbase, _RPW)],
                        out_hbm.at[pl.ds(c * _NP + rbase, _RPW)])

    return prop


_DEG_D = 128


def _make_deg():
    @functools.partial(
        pl.kernel,
        out_type=jax.ShapeDtypeStruct((_NC * _NP, _DEG_D), jnp.float32),
        mesh=_MESH,
        scratch_types=[
            pltpu.VMEM((_K,), jnp.int32),
            pltpu.VMEM((_K, _DEG_D), jnp.float32),
            pltpu.VMEM((_ZR, _DEG_D), jnp.float32),
            pltpu.VMEM_SHARED((_NP, _DEG_D), jnp.float32),
        ],
    )
    def deg(dst_hbm, out_hbm, idx_d, ones, zblk, acc):
        c = lax.axis_index("c")
        s = lax.axis_index("s")
        wid = c * _NS + s

        _fill(zblk, _ZR, _DEG_D, 0.0)
        _fill(ones, _K, _DEG_D, 1.0)
        rbase = s * _RPW
        for t in range(_RPW // _ZR):
            pltpu.sync_copy(zblk, acc.at[pl.ds(rbase + t * _ZR, _ZR)])
        plsc.subcore_barrier()

        ebase = wid * _EPW

        def body(j, carry):
            off = ebase + j * _K
            pltpu.sync_copy(dst_hbm.at[pl.ds(off, _K)], idx_d)
            pltpu.sync_copy(ones, acc.at[idx_d], add=True)
            return carry

        lax.fori_loop(0, _NCHUNK, body, 0)
        plsc.subcore_barrier()

        pltpu.sync_copy(acc.at[pl.ds(rbase, _RPW)],
                        out_hbm.at[pl.ds(c * _NP + rbase, _RPW)])

    return deg


_PROP = _make_prop(128)
_DEG = _make_deg()


def _dis(deg_ref):
    d0 = deg_ref[0:_N, 0:1]
    d1 = deg_ref[_NP:_NP + _N, 0:1]
    return lax.rsqrt(1.0 + d0 + d1)


def _acc_sum(acc_ref):
    return acc_ref[0:_N, :] + acc_ref[_NP:_NP + _N, :]


def _tc_first(h0_ref, w_ref, deg_ref, y_ref):
    dis = _dis(deg_ref)
    y_ref[...] = dis * jnp.dot(h0_ref[...], w_ref[...],
                               preferred_element_type=jnp.float32)


def _tc_mid(y_ref, acc_ref, deg_ref, w_ref, b_ref, out_ref):
    dis = _dis(deg_ref)
    tot = y_ref[...] + _acc_sum(acc_ref)
    h = jnp.tanh(dis * tot + b_ref[...])
    out_ref[...] = dis * jnp.dot(h, w_ref[...],
                                 preferred_element_type=jnp.float32)


def _tc_lat(y_ref, acc_ref, deg_ref, b_ref, condp_ref, out_ref):
    dis = _dis(deg_ref)
    t = dis * (y_ref[...] + _acc_sum(acc_ref)) + b_ref[...] + condp_ref[...]
    out_ref[...] = dis * t


def _tc_dec1(y_ref, acc_ref, deg_ref, w1_ref, b1_ref, w2_ref, out_ref):
    dis = _dis(deg_ref)
    p = dis * (y_ref[...] + _acc_sum(acc_ref))
    h = jnp.tanh(jnp.dot(p, w1_ref[...], preferred_element_type=jnp.float32)
                 + b1_ref[...])
    out_ref[...] = dis * jnp.dot(h, w2_ref[...],
                                 preferred_element_type=jnp.float32)


def _tc_last(y_ref, acc_ref, deg_ref, b_ref, out_ref):
    dis = _dis(deg_ref)
    out_ref[...] = (dis * (y_ref[...] + _acc_sum(acc_ref))
                    + b_ref[...])


def _tc(fn, out_dim, *args):
    return pl.pallas_call(
        fn, out_shape=jax.ShapeDtypeStruct((_N, out_dim), jnp.float32))(*args)


def kernel(feature, condition, edge_index, W_e1, b_e1, W_e2, b_e2, W_e3, b_e3,
           W_d1, b_d1, W_d2, b_d2, W_d3, b_d3):
    src = edge_index[0]
    dst = edge_index[1]
    h0 = jnp.concatenate([feature, condition], axis=1)
    b_e1r = b_e1.reshape(1, -1)
    b_e2r = b_e2.reshape(1, -1)
    b_d1r = b_d1.reshape(1, -1)
    b_d2r = b_d2.reshape(1, -1)
    b_d3r = b_d3.reshape(1, -1)
    W_e3p = jnp.pad(W_e3, ((0, 0), (0, 64)))
    b_e3p = jnp.pad(b_e3, (0, 64)).reshape(1, -1)
    W_d1p = jnp.pad(W_d1, ((0, 48), (0, 0)))
    condp = jnp.pad(condition, ((0, 0), (64, 48)))

    deg = _DEG(dst)
    y1 = _tc(_tc_first, 128, h0, W_e1, deg)
    a1 = _PROP(y1, src, dst)
    y2 = _tc(_tc_mid, 128, y1, a1, deg, W_e2, b_e1r)
    a2 = _PROP(y2, src, dst)
    y3 = _tc(_tc_mid, 128, y2, a2, deg, W_e3p, b_e2r)
    a3 = _PROP(y3, src, dst)
    y4 = _tc(_tc_lat, 128, y3, a3, deg, b_e3p, condp)
    a4 = _PROP(y4, src, dst)
    y5 = _tc(_tc_dec1, 128, y4, a4, deg, W_d1p, b_d1r, W_d2)
    a5 = _PROP(y5, src, dst)
    y6 = _tc(_tc_mid, 128, y5, a5, deg, W_d3, b_d2r)
    a6 = _PROP(y6, src, dst)
    out = _tc(_tc_last, 128, y6, a6, deg, b_d3r)
    return out
